# baseline (speedup 1.0000x reference)
import functools

import jax
import jax.numpy as jnp
from jax import lax
from jax.experimental import pallas as pl
from jax.experimental.pallas import tpu as pltpu
from jax.experimental.pallas import tpu_sc as plsc

N = 10000
NPAD = 10240
E = 320000
H = 128
G = 128
OUT = 128
CH = 128
ROWS = E // CH
NW = 32
TRIPS = 80
PROWS = NW * TRIPS
EP = PROWS * CH
RPT = NPAD // 16
JUNK_ROW = N + 16


def _sc_scatter(h, idx2d):
    mesh = plsc.VectorSubcoreMesh(core_axis_name="c", subcore_axis_name="s")

    @functools.partial(
        pl.kernel,
        mesh=mesh,
        out_type=jax.ShapeDtypeStruct((2, NPAD, H), jnp.float32),
        scratch_types=[
            pltpu.VMEM((TRIPS, CH), jnp.int32),
            pltpu.VMEM((CH, H), jnp.float32),
            pltpu.VMEM((CH, H), jnp.float32),
            pltpu.SemaphoreType.DMA,
            pltpu.SemaphoreType.DMA,
            pltpu.SemaphoreType.DMA,
            pltpu.SemaphoreType.DMA,
            pltpu.VMEM_SHARED((NPAD, H), jnp.float32),
        ],
    )
    def run(h_hbm, idx_hbm, out_hbm, idx_v, r0v, r1v,
            l0, l1, s0, s1, table_sh):
        cid = lax.axis_index("c")
        sid = lax.axis_index("s")
        wid = cid * 16 + sid
        start = pl.multiple_of(wid * TRIPS, 8)
        rows_v = (r0v, r1v)
        lsem = (l0, l1)
        ssem = (s0, s1)

        z16 = jnp.zeros((16,), jnp.float32)

        def zstore(r, carry):
            for c in range(H // 16):
                r0v[r, pl.ds(c * 16, 16)] = z16
            return carry

        lax.fori_loop(0, CH, zstore, 0)
        for k in range(RPT // CH):
            pltpu.async_copy(
                r0v, table_sh.at[pl.ds(sid * RPT + k * CH, CH)], l0)
        for k in range(RPT // CH):
            pltpu.make_async_copy(
                r0v, table_sh.at[pl.ds(sid * RPT + k * CH, CH)], l0).wait()

        pltpu.sync_copy(idx_hbm.at[pl.ds(start, TRIPS)], idx_v)
        plsc.subcore_barrier()

        def load_desc(t, b):
            row = jnp.minimum(start + t, ROWS - 1)
            return pltpu.make_async_copy(
                h_hbm.at[pl.ds(pl.multiple_of(row * CH, CH), CH)],
                rows_v[b], lsem[b])

        def scat_desc(t, b):
            return pltpu.make_async_copy(
                rows_v[b], table_sh.at[idx_v.at[t]], ssem[b])

        load_desc(0, 0).start()

        def pair(tt, carry):
            for b in range(2):
                t = tt * 2 + b
                load_desc(t, b).wait()
                pltpu.async_copy(
                    rows_v[b], table_sh.at[idx_v.at[t]], ssem[b], add=True)

                @pl.when(t >= 1)
                def _():
                    scat_desc(t - 1, 1 - b).wait()

                @pl.when(t + 1 < TRIPS)
                def _():
                    load_desc(t + 1, 1 - b).start()
            return carry

        lax.fori_loop(0, TRIPS // 2, pair, 0)
        scat_desc(TRIPS - 1, 1).wait()
        plsc.subcore_barrier()

        for k in range(RPT // CH):
            r0 = pl.multiple_of(sid * RPT + k * CH, CH)
            pltpu.async_copy(
                table_sh.at[pl.ds(r0, CH)], out_hbm.at[cid, pl.ds(r0, CH)], l0)
        for k in range(RPT // CH):
            r0 = pl.multiple_of(sid * RPT + k * CH, CH)
            pltpu.make_async_copy(
                table_sh.at[pl.ds(r0, CH)], out_hbm.at[cid, pl.ds(r0, CH)],
                l0).wait()

    return run(h, idx2d)


def _sc_gather(p, idx2d):
    mesh = plsc.VectorSubcoreMesh(core_axis_name="c", subcore_axis_name="s")

    @functools.partial(
        pl.kernel,
        mesh=mesh,
        out_type=jax.ShapeDtypeStruct((EP, H), jnp.float32),
        scratch_types=[
            pltpu.VMEM((TRIPS, CH), jnp.int32),
            pltpu.VMEM((CH, H), jnp.float32),
            pltpu.VMEM((CH, H), jnp.float32),
            pltpu.VMEM((CH, H), jnp.float32),
            pltpu.VMEM((CH, H), jnp.float32),
            pltpu.SemaphoreType.DMA,
            pltpu.SemaphoreType.DMA,
            pltpu.SemaphoreType.DMA,
            pltpu.SemaphoreType.DMA,
            pltpu.SemaphoreType.DMA,
            pltpu.SemaphoreType.DMA,
            pltpu.SemaphoreType.DMA,
            pltpu.SemaphoreType.DMA,
        ],
    )
    def run(p_hbm, idx_hbm, out_hbm, idx_v, r0v, r1v, r2v, r3v,
            g0, g1, g2, g3, w0, w1, w2, w3):
        cid = lax.axis_index("c")
        sid = lax.axis_index("s")
        wid = cid * 16 + sid
        start = pl.multiple_of(wid * TRIPS, 8)
        rows_v = (r0v, r1v, r2v, r3v)
        gsem = (g0, g1, g2, g3)
        wsem = (w0, w1, w2, w3)

        pltpu.sync_copy(idx_hbm.at[pl.ds(start, TRIPS)], idx_v)

        def gat_desc(t, b):
            return pltpu.make_async_copy(
                p_hbm.at[idx_v.at[t]], rows_v[b], gsem[b])

        def wb_desc(t, b):
            return pltpu.make_async_copy(
                rows_v[b],
                out_hbm.at[pl.ds(pl.multiple_of((start + t) * CH, CH), CH)],
                wsem[b])

        def quad(tt, carry):
            for b in range(4):
                t = tt * 4 + b

                @pl.when(t >= 4)
                def _():
                    wb_desc(t - 4, b).wait()

                gat_desc(t, b).start()

                @pl.when(t >= 1)
                def _():
                    gat_desc(t - 1, (b + 3) % 4).wait()
                    wb_desc(t - 1, (b + 3) % 4).start()
            return carry

        lax.fori_loop(0, TRIPS // 4, quad, 0)
        gat_desc(TRIPS - 1, 3).wait()
        wb_desc(TRIPS - 1, 3).start()
        for b in range(4):
            wb_desc(TRIPS - 4 + b, b).wait()

    return run(p, idx2d)


def _tc_p(parts, w2):
    def body(parts_ref, w2_ref, out_ref):
        psum = parts_ref[0] + parts_ref[1]
        out_ref[...] = lax.dot_general(
            psum, w2_ref[...], (((1,), (1,)), ((), ())),
            preferred_element_type=jnp.float32)

    return pl.pallas_call(
        body,
        out_shape=jax.ShapeDtypeStruct((NPAD, H), jnp.float32),
    )(parts, w2)


def _tc_step(h, h0, psrc, w2):
    nb = 500
    bs = E // nb

    def body(hrev_ref, h0_ref, psrc_ref, w2_ref, out_ref):
        q = lax.dot_general(
            hrev_ref[...], w2_ref[...], (((1,), (1,)), ((), ())),
            preferred_element_type=jnp.float32)
        out_ref[...] = jnp.maximum(h0_ref[...] + psrc_ref[...] - q, 0.0)

    return pl.pallas_call(
        body,
        grid=(nb,),
        in_specs=[
            pl.BlockSpec((bs, H), lambda i: ((i + nb // 2) % nb, 0)),
            pl.BlockSpec((bs, H), lambda i: (i, 0)),
            pl.BlockSpec((bs, H), lambda i: (i, 0)),
            pl.BlockSpec((H, H), lambda i: (0, 0)),
        ],
        out_specs=pl.BlockSpec((bs, H), lambda i: (i, 0)),
        out_shape=jax.ShapeDtypeStruct((E, H), jnp.float32),
    )(h, h0, psrc, w2)


def _tc_final(parts, x, batch2d, w3x, w3v, wh1, bh1, wh2, bh2):
    def body(parts_ref, x_ref, b_ref, w3x_ref, w3v_ref, wh1_ref, bh1_ref,
             wh2_ref, bh2_ref, out_ref):
        v = parts_ref[0, :N, :] + parts_ref[1, :N, :]
        na = jnp.maximum(
            lax.dot_general(x_ref[...], w3x_ref[...], (((1,), (1,)), ((), ())),
                            preferred_element_type=jnp.float32)
            + lax.dot_general(v, w3v_ref[...], (((1,), (1,)), ((), ())),
                              preferred_element_type=jnp.float32),
            0.0)
        gid = lax.broadcasted_iota(jnp.int32, (G, N), 0)
        onehot = (b_ref[...] == gid).astype(jnp.float32)
        g = lax.dot_general(onehot, na, (((1,), (0,)), ((), ())),
                            preferred_element_type=jnp.float32)
        t1 = jnp.maximum(
            lax.dot_general(g, wh1_ref[...], (((1,), (1,)), ((), ())),
                            preferred_element_type=jnp.float32)
            + bh1_ref[...], 0.0)
        out_ref[...] = lax.dot_general(
            t1, wh2_ref[...], (((1,), (1,)), ((), ())),
            preferred_element_type=jnp.float32) + bh2_ref[...]

    return pl.pallas_call(
        body,
        out_shape=jax.ShapeDtypeStruct((G, OUT), jnp.float32),
    )(parts, x, batch2d, w3x, w3v, wh1, bh1, wh2, bh2)


def kernel(x, edge_index, edge_attr, batch, W2, W3, Wh1, bh1, Wh2, bh2):
    src2d = edge_index[0].astype(jnp.int32).reshape(ROWS, CH)
    dst2d = edge_index[1].astype(jnp.int32).reshape(ROWS, CH)
    src = jnp.pad(src2d, ((0, PROWS - ROWS), (0, 0)))
    dst = jnp.pad(dst2d, ((0, PROWS - ROWS), (0, 0)),
                  constant_values=JUNK_ROW)
    h0 = edge_attr

    h = h0
    for _ in range(2):
        parts = _sc_scatter(h, dst)
        p = _tc_p(parts, W2)
        psrc = _sc_gather(p, src)
        h = _tc_step(h, h0, psrc, W2)

    parts = _sc_scatter(h, dst)
    out = _tc_final(
        parts, x, batch.astype(jnp.int32).reshape(1, N),
        W3[:, :H], W3[:, H:], Wh1, bh1.reshape(1, H), Wh2,
        bh2.reshape(1, OUT))
    return out


# speedup vs baseline: 1.4955x; 1.4955x over previous
import functools

import jax
import jax.numpy as jnp
from jax import lax
from jax.experimental import pallas as pl
from jax.experimental.pallas import tpu as pltpu
from jax.experimental.pallas import tpu_sc as plsc

N = 10000
NPAD = 10240
E = 320000
H = 128
G = 128
OUT = 128
CH = 128
ROWS = E // CH
NW = 32
TRIPS = 80
PROWS = NW * TRIPS
EP = PROWS * CH
RPT = NPAD // 16
JUNK_ROW = N + 16


def _sc_scatter(h, idx2d):
    mesh = plsc.VectorSubcoreMesh(core_axis_name="c", subcore_axis_name="s")

    @functools.partial(
        pl.kernel,
        mesh=mesh,
        out_type=jax.ShapeDtypeStruct((2, NPAD, H), jnp.float32),
        scratch_types=[
            pltpu.VMEM((TRIPS, CH), jnp.int32),
            pltpu.VMEM((CH, H), jnp.float32),
            pltpu.VMEM((CH, H), jnp.float32),
            pltpu.SemaphoreType.DMA,
            pltpu.SemaphoreType.DMA,
            pltpu.SemaphoreType.DMA,
            pltpu.SemaphoreType.DMA,
            pltpu.VMEM_SHARED((NPAD, H), jnp.float32),
        ],
    )
    def run(h_hbm, idx_hbm, out_hbm, idx_v, r0v, r1v,
            l0, l1, s0, s1, table_sh):
        cid = lax.axis_index("c")
        sid = lax.axis_index("s")
        wid = cid * 16 + sid
        start = pl.multiple_of(wid * TRIPS, 8)
        rows_v = (r0v, r1v)
        lsem = (l0, l1)
        ssem = (s0, s1)

        z16 = jnp.zeros((16,), jnp.float32)

        def zstore(r, carry):
            for c in range(H // 16):
                r0v[r, pl.ds(c * 16, 16)] = z16
            return carry

        lax.fori_loop(0, CH, zstore, 0)
        for k in range(RPT // CH):
            pltpu.async_copy(
                r0v, table_sh.at[pl.ds(sid * RPT + k * CH, CH)], l0)
        for k in range(RPT // CH):
            pltpu.make_async_copy(
                r0v, table_sh.at[pl.ds(sid * RPT + k * CH, CH)], l0).wait()

        pltpu.sync_copy(idx_hbm.at[pl.ds(start, TRIPS)], idx_v)
        plsc.subcore_barrier()

        def load_desc(t, b):
            row = jnp.minimum(start + t, ROWS - 1)
            return pltpu.make_async_copy(
                h_hbm.at[pl.ds(pl.multiple_of(row * CH, CH), CH)],
                rows_v[b], lsem[b])

        def scat_desc(t, b):
            return pltpu.make_async_copy(
                rows_v[b], table_sh.at[idx_v.at[t]], ssem[b])

        load_desc(0, 0).start()

        def pair(tt, carry):
            for b in range(2):
                t = tt * 2 + b
                load_desc(t, b).wait()
                pltpu.async_copy(
                    rows_v[b], table_sh.at[idx_v.at[t]], ssem[b], add=True)

                @pl.when(t >= 1)
                def _():
                    scat_desc(t - 1, 1 - b).wait()

                @pl.when(t + 1 < TRIPS)
                def _():
                    load_desc(t + 1, 1 - b).start()
            return carry

        lax.fori_loop(0, TRIPS // 2, pair, 0)
        scat_desc(TRIPS - 1, 1).wait()
        plsc.subcore_barrier()

        for k in range(RPT // CH):
            r0 = pl.multiple_of(sid * RPT + k * CH, CH)
            pltpu.async_copy(
                table_sh.at[pl.ds(r0, CH)], out_hbm.at[cid, pl.ds(r0, CH)], l0)
        for k in range(RPT // CH):
            r0 = pl.multiple_of(sid * RPT + k * CH, CH)
            pltpu.make_async_copy(
                table_sh.at[pl.ds(r0, CH)], out_hbm.at[cid, pl.ds(r0, CH)],
                l0).wait()

    return run(h, idx2d)


def _sc_gather(p, idx2d):
    mesh = plsc.VectorSubcoreMesh(core_axis_name="c", subcore_axis_name="s")

    @functools.partial(
        pl.kernel,
        mesh=mesh,
        out_type=jax.ShapeDtypeStruct((EP, H), jnp.float32),
        scratch_types=[
            pltpu.VMEM((TRIPS, CH), jnp.int32),
            pltpu.VMEM((CH, H), jnp.float32),
            pltpu.VMEM((CH, H), jnp.float32),
            pltpu.VMEM((CH, H), jnp.float32),
            pltpu.VMEM((CH, H), jnp.float32),
            pltpu.SemaphoreType.DMA,
            pltpu.SemaphoreType.DMA,
            pltpu.SemaphoreType.DMA,
            pltpu.SemaphoreType.DMA,
            pltpu.SemaphoreType.DMA,
            pltpu.SemaphoreType.DMA,
            pltpu.SemaphoreType.DMA,
            pltpu.SemaphoreType.DMA,
        ],
    )
    def run(p_hbm, idx_hbm, out_hbm, idx_v, r0v, r1v, r2v, r3v,
            g0, g1, g2, g3, w0, w1, w2, w3):
        cid = lax.axis_index("c")
        sid = lax.axis_index("s")
        wid = cid * 16 + sid
        start = pl.multiple_of(wid * TRIPS, 8)
        rows_v = (r0v, r1v, r2v, r3v)
        gsem = (g0, g1, g2, g3)
        wsem = (w0, w1, w2, w3)

        pltpu.sync_copy(idx_hbm.at[pl.ds(start, TRIPS)], idx_v)

        def gat_desc(t, b):
            return pltpu.make_async_copy(
                p_hbm.at[idx_v.at[t]], rows_v[b], gsem[b])

        def wb_desc(t, b):
            return pltpu.make_async_copy(
                rows_v[b],
                out_hbm.at[pl.ds(pl.multiple_of((start + t) * CH, CH), CH)],
                wsem[b])

        def quad(tt, carry):
            for b in range(4):
                t = tt * 4 + b

                @pl.when(t >= 4)
                def _():
                    wb_desc(t - 4, b).wait()

                gat_desc(t, b).start()

                @pl.when(t >= 1)
                def _():
                    gat_desc(t - 1, (b + 3) % 4).wait()
                    wb_desc(t - 1, (b + 3) % 4).start()
            return carry

        lax.fori_loop(0, TRIPS // 4, quad, 0)
        gat_desc(TRIPS - 1, 3).wait()
        wb_desc(TRIPS - 1, 3).start()
        for b in range(4):
            wb_desc(TRIPS - 4 + b, b).wait()

    return run(p, idx2d)


def _tc_p(parts, w2):
    def body(parts_ref, w2_ref, out_ref):
        psum = parts_ref[0] + parts_ref[1]
        out_ref[...] = lax.dot_general(
            psum, w2_ref[...], (((1,), (1,)), ((), ())),
            preferred_element_type=jnp.float32)

    return pl.pallas_call(
        body,
        out_shape=jax.ShapeDtypeStruct((NPAD, H), jnp.float32),
    )(parts, w2)


def _tc_step(h, h0, psrc, w2):
    nb = 500
    bs = E // nb

    def body(hrev_ref, h0_ref, psrc_ref, w2_ref, out_ref):
        q = lax.dot_general(
            hrev_ref[...], w2_ref[...], (((1,), (1,)), ((), ())),
            preferred_element_type=jnp.float32)
        out_ref[...] = jnp.maximum(h0_ref[...] + psrc_ref[...] - q, 0.0)

    return pl.pallas_call(
        body,
        grid=(nb,),
        in_specs=[
            pl.BlockSpec((bs, H), lambda i: ((i + nb // 2) % nb, 0)),
            pl.BlockSpec((bs, H), lambda i: (i, 0)),
            pl.BlockSpec((bs, H), lambda i: (i, 0)),
            pl.BlockSpec((H, H), lambda i: (0, 0)),
        ],
        out_specs=pl.BlockSpec((bs, H), lambda i: (i, 0)),
        out_shape=jax.ShapeDtypeStruct((E, H), jnp.float32),
    )(h, h0, psrc, w2)


def _tc_final(parts, x, batch2d, w3x, w3v, wh1, bh1, wh2, bh2):
    def body(parts_ref, x_ref, b_ref, w3x_ref, w3v_ref, wh1_ref, bh1_ref,
             wh2_ref, bh2_ref, out_ref):
        v = parts_ref[0, :N, :] + parts_ref[1, :N, :]
        na = jnp.maximum(
            lax.dot_general(x_ref[...], w3x_ref[...], (((1,), (1,)), ((), ())),
                            preferred_element_type=jnp.float32)
            + lax.dot_general(v, w3v_ref[...], (((1,), (1,)), ((), ())),
                              preferred_element_type=jnp.float32),
            0.0)
        gid = lax.broadcasted_iota(jnp.int32, (G, N), 0)
        onehot = (b_ref[...] == gid).astype(jnp.float32)
        g = lax.dot_general(onehot, na, (((1,), (0,)), ((), ())),
                            preferred_element_type=jnp.float32)
        t1 = jnp.maximum(
            lax.dot_general(g, wh1_ref[...], (((1,), (1,)), ((), ())),
                            preferred_element_type=jnp.float32)
            + bh1_ref[...], 0.0)
        out_ref[...] = lax.dot_general(
            t1, wh2_ref[...], (((1,), (1,)), ((), ())),
            preferred_element_type=jnp.float32) + bh2_ref[...]

    return pl.pallas_call(
        body,
        out_shape=jax.ShapeDtypeStruct((G, OUT), jnp.float32),
    )(parts, x, batch2d, w3x, w3v, wh1, bh1, wh2, bh2)


def kernel(x, edge_index, edge_attr, batch, W2, W3, Wh1, bh1, Wh2, bh2):
    src2d = edge_index[0].astype(jnp.int32).reshape(ROWS, CH)
    dst2d = edge_index[1].astype(jnp.int32).reshape(ROWS, CH)
    spread = (jnp.arange(PROWS * CH, dtype=jnp.int32) % N).reshape(PROWS, CH)
    src = jnp.where(
        (jnp.arange(PROWS, dtype=jnp.int32) < ROWS)[:, None],
        jnp.pad(src2d, ((0, PROWS - ROWS), (0, 0))), spread)
    dst = jnp.pad(dst2d, ((0, PROWS - ROWS), (0, 0)),
                  constant_values=JUNK_ROW)
    h0 = edge_attr

    h = h0
    for _ in range(2):
        parts = _sc_scatter(h, dst)
        p = _tc_p(parts, W2)
        psrc = _sc_gather(p, src)
        h = _tc_step(h, h0, psrc, W2)

    parts = _sc_scatter(h, dst)
    out = _tc_final(
        parts, x, batch.astype(jnp.int32).reshape(1, N),
        W3[:, :H], W3[:, H:], Wh1, bh1.reshape(1, H), Wh2,
        bh2.reshape(1, OUT))
    return out


# speedup vs baseline: 1.8235x; 1.2193x over previous
import functools

import jax
import jax.numpy as jnp
from jax import lax
from jax.experimental import pallas as pl
from jax.experimental.pallas import tpu as pltpu
from jax.experimental.pallas import tpu_sc as plsc

N = 10000
NPAD = 10240
E = 320000
H = 128
G = 128
OUT = 128
CH = 128
ROWS = E // CH
NW = 32
TRIPS = 80
PROWS = NW * TRIPS
EP = PROWS * CH
RPT = NPAD // 16
JUNK_ROW = N + 16


def _sc_scatter(h, idx2d):
    mesh = plsc.VectorSubcoreMesh(core_axis_name="c", subcore_axis_name="s")

    @functools.partial(
        pl.kernel,
        mesh=mesh,
        out_type=jax.ShapeDtypeStruct((2, NPAD, H), jnp.float32),
        scratch_types=[
            pltpu.VMEM((TRIPS, CH), jnp.int32),
            pltpu.VMEM((CH, H), jnp.float32),
            pltpu.VMEM((CH, H), jnp.float32),
            pltpu.SemaphoreType.DMA,
            pltpu.SemaphoreType.DMA,
            pltpu.SemaphoreType.DMA,
            pltpu.SemaphoreType.DMA,
            pltpu.VMEM_SHARED((NPAD, H), jnp.float32),
        ],
    )
    def run(h_hbm, idx_hbm, out_hbm, idx_v, r0v, r1v,
            l0, l1, s0, s1, table_sh):
        cid = lax.axis_index("c")
        sid = lax.axis_index("s")
        wid = cid * 16 + sid
        start = pl.multiple_of(wid * TRIPS, 8)
        rows_v = (r0v, r1v)
        lsem = (l0, l1)
        ssem = (s0, s1)

        z16 = jnp.zeros((16,), jnp.float32)

        def zstore(r, carry):
            for c in range(H // 16):
                r0v[r, pl.ds(c * 16, 16)] = z16
            return carry

        lax.fori_loop(0, CH, zstore, 0)
        for k in range(RPT // CH):
            pltpu.async_copy(
                r0v, table_sh.at[pl.ds(sid * RPT + k * CH, CH)], l0)
        for k in range(RPT // CH):
            pltpu.make_async_copy(
                r0v, table_sh.at[pl.ds(sid * RPT + k * CH, CH)], l0).wait()

        pltpu.sync_copy(idx_hbm.at[pl.ds(start, TRIPS)], idx_v)
        plsc.subcore_barrier()

        def load_desc(t, b):
            row = jnp.minimum(start + t, ROWS - 1)
            return pltpu.make_async_copy(
                h_hbm.at[pl.ds(pl.multiple_of(row * CH, CH), CH)],
                rows_v[b], lsem[b])

        def scat_desc(t, b):
            return pltpu.make_async_copy(
                rows_v[b], table_sh.at[idx_v.at[t]], ssem[b])

        load_desc(0, 0).start()

        def pair(tt, carry):
            for b in range(2):
                t = tt * 2 + b
                load_desc(t, b).wait()
                pltpu.async_copy(
                    rows_v[b], table_sh.at[idx_v.at[t]], ssem[b], add=True)

                @pl.when(t >= 1)
                def _():
                    scat_desc(t - 1, 1 - b).wait()

                @pl.when(t + 1 < TRIPS)
                def _():
                    load_desc(t + 1, 1 - b).start()
            return carry

        lax.fori_loop(0, TRIPS // 2, pair, 0)
        scat_desc(TRIPS - 1, 1).wait()
        plsc.subcore_barrier()

        for k in range(RPT // CH):
            r0 = pl.multiple_of(sid * RPT + k * CH, CH)
            pltpu.async_copy(
                table_sh.at[pl.ds(r0, CH)], out_hbm.at[cid, pl.ds(r0, CH)], l0)
        for k in range(RPT // CH):
            r0 = pl.multiple_of(sid * RPT + k * CH, CH)
            pltpu.make_async_copy(
                table_sh.at[pl.ds(r0, CH)], out_hbm.at[cid, pl.ds(r0, CH)],
                l0).wait()

    return run(h, idx2d)


def _sc_gather(p, idx2d):
    mesh = plsc.VectorSubcoreMesh(core_axis_name="c", subcore_axis_name="s")

    @functools.partial(
        pl.kernel,
        mesh=mesh,
        out_type=jax.ShapeDtypeStruct((EP, H), jnp.float32),
        scratch_types=[
            pltpu.VMEM((TRIPS, CH), jnp.int32),
            pltpu.VMEM((CH, H), jnp.float32),
            pltpu.VMEM((CH, H), jnp.float32),
            pltpu.VMEM((CH, H), jnp.float32),
            pltpu.VMEM((CH, H), jnp.float32),
            pltpu.SemaphoreType.DMA,
            pltpu.SemaphoreType.DMA,
            pltpu.SemaphoreType.DMA,
            pltpu.SemaphoreType.DMA,
            pltpu.SemaphoreType.DMA,
            pltpu.SemaphoreType.DMA,
            pltpu.SemaphoreType.DMA,
            pltpu.SemaphoreType.DMA,
        ],
    )
    def run(p_hbm, idx_hbm, out_hbm, idx_v, r0v, r1v, r2v, r3v,
            g0, g1, g2, g3, w0, w1, w2, w3):
        cid = lax.axis_index("c")
        sid = lax.axis_index("s")
        wid = cid * 16 + sid
        start = pl.multiple_of(wid * TRIPS, 8)
        rows_v = (r0v, r1v, r2v, r3v)
        gsem = (g0, g1, g2, g3)
        wsem = (w0, w1, w2, w3)

        pltpu.sync_copy(idx_hbm.at[pl.ds(start, TRIPS)], idx_v)

        def gat_desc(t, b):
            return pltpu.make_async_copy(
                p_hbm.at[idx_v.at[t]], rows_v[b], gsem[b])

        def wb_desc(t, b):
            return pltpu.make_async_copy(
                rows_v[b],
                out_hbm.at[pl.ds(pl.multiple_of((start + t) * CH, CH), CH)],
                wsem[b])

        def quad(tt, carry):
            for b in range(4):
                t = tt * 4 + b

                @pl.when(t >= 4)
                def _():
                    wb_desc(t - 4, b).wait()

                gat_desc(t, b).start()

                @pl.when(t >= 1)
                def _():
                    gat_desc(t - 1, (b + 3) % 4).wait()
                    wb_desc(t - 1, (b + 3) % 4).start()
            return carry

        lax.fori_loop(0, TRIPS // 4, quad, 0)
        gat_desc(TRIPS - 1, 3).wait()
        wb_desc(TRIPS - 1, 3).start()
        for b in range(4):
            wb_desc(TRIPS - 4 + b, b).wait()

    return run(p, idx2d)


def _tc_p(parts, w2):
    def body(parts_ref, w2_ref, out_ref):
        psum = parts_ref[0] + parts_ref[1]
        out_ref[...] = lax.dot_general(
            psum, w2_ref[...], (((1,), (1,)), ((), ())),
            preferred_element_type=jnp.float32)

    return pl.pallas_call(
        body,
        out_shape=jax.ShapeDtypeStruct((NPAD, H), jnp.float32),
    )(parts, w2)


def _tc_step(h, h0, psrc, w2):
    nb = 250
    bs = E // nb

    def body(hrev_ref, h0_ref, psrc_ref, w2_ref, out_ref):
        q = lax.dot_general(
            hrev_ref[...], w2_ref[...], (((1,), (1,)), ((), ())),
            preferred_element_type=jnp.float32)
        out_ref[...] = jnp.maximum(h0_ref[...] + psrc_ref[...] - q, 0.0)

    return pl.pallas_call(
        body,
        grid=(nb,),
        in_specs=[
            pl.BlockSpec((bs, H), lambda i: ((i + nb // 2) % nb, 0)),
            pl.BlockSpec((bs, H), lambda i: (i, 0)),
            pl.BlockSpec((bs, H), lambda i: (i, 0)),
            pl.BlockSpec((H, H), lambda i: (0, 0)),
        ],
        out_specs=pl.BlockSpec((bs, H), lambda i: (i, 0)),
        out_shape=jax.ShapeDtypeStruct((E, H), jnp.float32),
    )(h, h0, psrc, w2)


def _tc_final(parts, x, batch2d, w3x, w3v, wh1, bh1, wh2, bh2):
    def body(parts_ref, x_ref, b_ref, w3x_ref, w3v_ref, wh1_ref, bh1_ref,
             wh2_ref, bh2_ref, out_ref):
        v = parts_ref[0, :N, :] + parts_ref[1, :N, :]
        na = jnp.maximum(
            lax.dot_general(x_ref[...], w3x_ref[...], (((1,), (1,)), ((), ())),
                            preferred_element_type=jnp.float32)
            + lax.dot_general(v, w3v_ref[...], (((1,), (1,)), ((), ())),
                              preferred_element_type=jnp.float32),
            0.0)
        gid = lax.broadcasted_iota(jnp.int32, (G, N), 0)
        onehot = (b_ref[...] == gid).astype(jnp.float32)
        g = lax.dot_general(onehot, na, (((1,), (0,)), ((), ())),
                            preferred_element_type=jnp.float32)
        t1 = jnp.maximum(
            lax.dot_general(g, wh1_ref[...], (((1,), (1,)), ((), ())),
                            preferred_element_type=jnp.float32)
            + bh1_ref[...], 0.0)
        out_ref[...] = lax.dot_general(
            t1, wh2_ref[...], (((1,), (1,)), ((), ())),
            preferred_element_type=jnp.float32) + bh2_ref[...]

    return pl.pallas_call(
        body,
        out_shape=jax.ShapeDtypeStruct((G, OUT), jnp.float32),
    )(parts, x, batch2d, w3x, w3v, wh1, bh1, wh2, bh2)


def kernel(x, edge_index, edge_attr, batch, W2, W3, Wh1, bh1, Wh2, bh2):
    src2d = edge_index[0].astype(jnp.int32).reshape(ROWS, CH)
    dst2d = edge_index[1].astype(jnp.int32).reshape(ROWS, CH)
    spread = (jnp.arange(PROWS * CH, dtype=jnp.int32) % N).reshape(PROWS, CH)
    src = jnp.where(
        (jnp.arange(PROWS, dtype=jnp.int32) < ROWS)[:, None],
        jnp.pad(src2d, ((0, PROWS - ROWS), (0, 0))), spread)
    dst = jnp.pad(dst2d, ((0, PROWS - ROWS), (0, 0)),
                  constant_values=JUNK_ROW)
    h0 = edge_attr

    h = h0
    for _ in range(2):
        parts = _sc_scatter(h, dst)
        p = _tc_p(parts, W2)
        psrc = _sc_gather(p, src)
        h = _tc_step(h, h0, psrc, W2)

    parts = _sc_scatter(h, dst)
    out = _tc_final(
        parts, x, batch.astype(jnp.int32).reshape(1, N),
        W3[:, :H], W3[:, H:], Wh1, bh1.reshape(1, H), Wh2,
        bh2.reshape(1, OUT))
    return out


# speedup vs baseline: 2.0874x; 1.1447x over previous
import functools

import jax
import jax.numpy as jnp
from jax import lax
from jax.experimental import pallas as pl
from jax.experimental.pallas import tpu as pltpu
from jax.experimental.pallas import tpu_sc as plsc

N = 10000
NPAD = 10240
E = 320000
H = 128
G = 128
OUT = 128
CH = 128
ROWS = E // CH
NW = 32
TRIPS = 80
PROWS = NW * TRIPS
EP = PROWS * CH
RPT = NPAD // 16
JUNK_ROW = N + 16


def _sc_scatter(h, idx2d):
    mesh = plsc.VectorSubcoreMesh(core_axis_name="c", subcore_axis_name="s")

    @functools.partial(
        pl.kernel,
        mesh=mesh,
        out_type=jax.ShapeDtypeStruct((2, NPAD, H), jnp.float32),
        scratch_types=[
            pltpu.VMEM((TRIPS, CH), jnp.int32),
            pltpu.VMEM((CH, H), jnp.float32),
            pltpu.VMEM((CH, H), jnp.float32),
            pltpu.SemaphoreType.DMA,
            pltpu.SemaphoreType.DMA,
            pltpu.SemaphoreType.DMA,
            pltpu.SemaphoreType.DMA,
            pltpu.VMEM_SHARED((NPAD, H), jnp.float32),
        ],
    )
    def run(h_hbm, idx_hbm, out_hbm, idx_v, r0v, r1v,
            l0, l1, s0, s1, table_sh):
        cid = lax.axis_index("c")
        sid = lax.axis_index("s")
        wid = cid * 16 + sid
        start = pl.multiple_of(wid * TRIPS, 8)
        rows_v = (r0v, r1v)
        lsem = (l0, l1)
        ssem = (s0, s1)

        z16 = jnp.zeros((16,), jnp.float32)

        def zstore(r, carry):
            for c in range(H // 16):
                r0v[r, pl.ds(c * 16, 16)] = z16
            return carry

        lax.fori_loop(0, CH, zstore, 0)
        for k in range(RPT // CH):
            pltpu.async_copy(
                r0v, table_sh.at[pl.ds(sid * RPT + k * CH, CH)], l0)
        for k in range(RPT // CH):
            pltpu.make_async_copy(
                r0v, table_sh.at[pl.ds(sid * RPT + k * CH, CH)], l0).wait()

        pltpu.sync_copy(idx_hbm.at[pl.ds(start, TRIPS)], idx_v)
        plsc.subcore_barrier()

        def load_desc(t, b):
            row = jnp.minimum(start + t, ROWS - 1)
            return pltpu.make_async_copy(
                h_hbm.at[pl.ds(pl.multiple_of(row * CH, CH), CH)],
                rows_v[b], lsem[b])

        def scat_desc(t, b):
            return pltpu.make_async_copy(
                rows_v[b], table_sh.at[idx_v.at[t]], ssem[b])

        load_desc(0, 0).start()

        def pair(tt, carry):
            for b in range(2):
                t = tt * 2 + b
                load_desc(t, b).wait()
                pltpu.async_copy(
                    rows_v[b], table_sh.at[idx_v.at[t]], ssem[b], add=True)

                @pl.when(t >= 1)
                def _():
                    scat_desc(t - 1, 1 - b).wait()

                @pl.when(t + 1 < TRIPS)
                def _():
                    load_desc(t + 1, 1 - b).start()
            return carry

        lax.fori_loop(0, TRIPS // 2, pair, 0)
        scat_desc(TRIPS - 1, 1).wait()
        plsc.subcore_barrier()

        for k in range(RPT // CH):
            r0 = pl.multiple_of(sid * RPT + k * CH, CH)
            pltpu.async_copy(
                table_sh.at[pl.ds(r0, CH)], out_hbm.at[cid, pl.ds(r0, CH)], l0)
        for k in range(RPT // CH):
            r0 = pl.multiple_of(sid * RPT + k * CH, CH)
            pltpu.make_async_copy(
                table_sh.at[pl.ds(r0, CH)], out_hbm.at[cid, pl.ds(r0, CH)],
                l0).wait()

    return run(h, idx2d)


def _sc_gather(p, idx2d):
    mesh = plsc.VectorSubcoreMesh(core_axis_name="c", subcore_axis_name="s")

    @functools.partial(
        pl.kernel,
        mesh=mesh,
        out_type=jax.ShapeDtypeStruct((EP, H), jnp.float32),
        scratch_types=[
            pltpu.VMEM((TRIPS, CH), jnp.int32),
            pltpu.VMEM((CH, H), jnp.float32),
            pltpu.VMEM((CH, H), jnp.float32),
            pltpu.VMEM((CH, H), jnp.float32),
            pltpu.VMEM((CH, H), jnp.float32),
            pltpu.SemaphoreType.DMA,
            pltpu.SemaphoreType.DMA,
            pltpu.SemaphoreType.DMA,
            pltpu.SemaphoreType.DMA,
            pltpu.SemaphoreType.DMA,
            pltpu.SemaphoreType.DMA,
            pltpu.SemaphoreType.DMA,
            pltpu.SemaphoreType.DMA,
        ],
    )
    def run(p_hbm, idx_hbm, out_hbm, idx_v, r0v, r1v, r2v, r3v,
            g0, g1, g2, g3, w0, w1, w2, w3):
        cid = lax.axis_index("c")
        sid = lax.axis_index("s")
        wid = cid * 16 + sid
        start = pl.multiple_of(wid * TRIPS, 8)
        rows_v = (r0v, r1v, r2v, r3v)
        gsem = (g0, g1, g2, g3)
        wsem = (w0, w1, w2, w3)

        pltpu.sync_copy(idx_hbm.at[pl.ds(start, TRIPS)], idx_v)

        def gat_desc(t, b):
            return pltpu.make_async_copy(
                p_hbm.at[idx_v.at[t]], rows_v[b], gsem[b])

        def wb_desc(t, b):
            return pltpu.make_async_copy(
                rows_v[b],
                out_hbm.at[pl.ds(pl.multiple_of((start + t) * CH, CH), CH)],
                wsem[b])

        def quad(tt, carry):
            for b in range(4):
                t = tt * 4 + b

                @pl.when(t >= 4)
                def _():
                    wb_desc(t - 4, b).wait()

                gat_desc(t, b).start()

                @pl.when(t >= 1)
                def _():
                    gat_desc(t - 1, (b + 3) % 4).wait()
                    wb_desc(t - 1, (b + 3) % 4).start()
            return carry

        lax.fori_loop(0, TRIPS // 4, quad, 0)
        gat_desc(TRIPS - 1, 3).wait()
        wb_desc(TRIPS - 1, 3).start()
        for b in range(4):
            wb_desc(TRIPS - 4 + b, b).wait()

    return run(p, idx2d)


def _tc_p(parts, w2):
    def body(parts_ref, w2_ref, out_ref):
        psum = parts_ref[0] + parts_ref[1]
        out_ref[...] = lax.dot_general(
            psum, w2_ref[...], (((1,), (1,)), ((), ())),
            preferred_element_type=jnp.float32)

    return pl.pallas_call(
        body,
        out_shape=jax.ShapeDtypeStruct((NPAD, H), jnp.float32),
    )(parts, w2)


def _tc_step(h, h0, psrc, w2):
    nb = 100
    bs = E // nb

    def body(hrev_ref, h0_ref, psrc_ref, w2_ref, out_ref):
        q = lax.dot_general(
            hrev_ref[...], w2_ref[...], (((1,), (1,)), ((), ())),
            preferred_element_type=jnp.float32)
        out_ref[...] = jnp.maximum(h0_ref[...] + psrc_ref[...] - q, 0.0)

    return pl.pallas_call(
        body,
        grid=(nb,),
        in_specs=[
            pl.BlockSpec((bs, H), lambda i: ((i + nb // 2) % nb, 0)),
            pl.BlockSpec((bs, H), lambda i: (i, 0)),
            pl.BlockSpec((bs, H), lambda i: (i, 0)),
            pl.BlockSpec((H, H), lambda i: (0, 0)),
        ],
        out_specs=pl.BlockSpec((bs, H), lambda i: (i, 0)),
        out_shape=jax.ShapeDtypeStruct((E, H), jnp.float32),
    )(h, h0, psrc, w2)


def _tc_final(parts, x, batch2d, w3x, w3v, wh1, bh1, wh2, bh2):
    def body(parts_ref, x_ref, b_ref, w3x_ref, w3v_ref, wh1_ref, bh1_ref,
             wh2_ref, bh2_ref, out_ref):
        v = parts_ref[0, :N, :] + parts_ref[1, :N, :]
        na = jnp.maximum(
            lax.dot_general(x_ref[...], w3x_ref[...], (((1,), (1,)), ((), ())),
                            preferred_element_type=jnp.float32)
            + lax.dot_general(v, w3v_ref[...], (((1,), (1,)), ((), ())),
                              preferred_element_type=jnp.float32),
            0.0)
        gid = lax.broadcasted_iota(jnp.int32, (G, N), 0)
        onehot = (b_ref[...] == gid).astype(jnp.float32)
        g = lax.dot_general(onehot, na, (((1,), (0,)), ((), ())),
                            preferred_element_type=jnp.float32)
        t1 = jnp.maximum(
            lax.dot_general(g, wh1_ref[...], (((1,), (1,)), ((), ())),
                            preferred_element_type=jnp.float32)
            + bh1_ref[...], 0.0)
        out_ref[...] = lax.dot_general(
            t1, wh2_ref[...], (((1,), (1,)), ((), ())),
            preferred_element_type=jnp.float32) + bh2_ref[...]

    return pl.pallas_call(
        body,
        out_shape=jax.ShapeDtypeStruct((G, OUT), jnp.float32),
    )(parts, x, batch2d, w3x, w3v, wh1, bh1, wh2, bh2)


def kernel(x, edge_index, edge_attr, batch, W2, W3, Wh1, bh1, Wh2, bh2):
    src2d = edge_index[0].astype(jnp.int32).reshape(ROWS, CH)
    dst2d = edge_index[1].astype(jnp.int32).reshape(ROWS, CH)
    spread = (jnp.arange(PROWS * CH, dtype=jnp.int32) % N).reshape(PROWS, CH)
    src = jnp.where(
        (jnp.arange(PROWS, dtype=jnp.int32) < ROWS)[:, None],
        jnp.pad(src2d, ((0, PROWS - ROWS), (0, 0))), spread)
    dst = jnp.pad(dst2d, ((0, PROWS - ROWS), (0, 0)),
                  constant_values=JUNK_ROW)
    h0 = edge_attr

    h = h0
    for _ in range(2):
        parts = _sc_scatter(h, dst)
        p = _tc_p(parts, W2)
        psrc = _sc_gather(p, src)
        h = _tc_step(h, h0, psrc, W2)

    parts = _sc_scatter(h, dst)
    out = _tc_final(
        parts, x, batch.astype(jnp.int32).reshape(1, N),
        W3[:, :H], W3[:, H:], Wh1, bh1.reshape(1, H), Wh2,
        bh2.reshape(1, OUT))
    return out


# speedup vs baseline: 2.1101x; 1.0109x over previous
import functools

import jax
import jax.numpy as jnp
from jax import lax
from jax.experimental import pallas as pl
from jax.experimental.pallas import tpu as pltpu
from jax.experimental.pallas import tpu_sc as plsc

N = 10000
NPAD = 10240
E = 320000
H = 128
G = 128
OUT = 128
CH = 128
ROWS = E // CH
NW = 32
TRIPS = 80
PROWS = NW * TRIPS
EP = PROWS * CH
RPT = NPAD // 16
JUNK_ROW = N + 16


def _sc_scatter(h, idx2d):
    mesh = plsc.VectorSubcoreMesh(core_axis_name="c", subcore_axis_name="s")

    @functools.partial(
        pl.kernel,
        mesh=mesh,
        out_type=jax.ShapeDtypeStruct((2, NPAD, H), jnp.float32),
        scratch_types=[
            pltpu.VMEM((TRIPS, CH), jnp.int32),
            pltpu.VMEM((CH, H), jnp.float32),
            pltpu.VMEM((CH, H), jnp.float32),
            pltpu.SemaphoreType.DMA,
            pltpu.SemaphoreType.DMA,
            pltpu.SemaphoreType.DMA,
            pltpu.SemaphoreType.DMA,
            pltpu.VMEM_SHARED((NPAD, H), jnp.float32),
        ],
    )
    def run(h_hbm, idx_hbm, out_hbm, idx_v, r0v, r1v,
            l0, l1, s0, s1, table_sh):
        cid = lax.axis_index("c")
        sid = lax.axis_index("s")
        wid = cid * 16 + sid
        start = pl.multiple_of(wid * TRIPS, 8)
        rows_v = (r0v, r1v)
        lsem = (l0, l1)
        ssem = (s0, s1)

        z16 = jnp.zeros((16,), jnp.float32)

        def zstore(r, carry):
            for c in range(H // 16):
                r0v[r, pl.ds(c * 16, 16)] = z16
            return carry

        lax.fori_loop(0, CH, zstore, 0)
        for k in range(RPT // CH):
            pltpu.async_copy(
                r0v, table_sh.at[pl.ds(sid * RPT + k * CH, CH)], l0)
        for k in range(RPT // CH):
            pltpu.make_async_copy(
                r0v, table_sh.at[pl.ds(sid * RPT + k * CH, CH)], l0).wait()

        pltpu.sync_copy(idx_hbm.at[pl.ds(start, TRIPS)], idx_v)
        plsc.subcore_barrier()

        def load_desc(t, b):
            row = jnp.minimum(start + t, ROWS - 1)
            return pltpu.make_async_copy(
                h_hbm.at[pl.ds(pl.multiple_of(row * CH, CH), CH)],
                rows_v[b], lsem[b])

        def scat_desc(t, b):
            return pltpu.make_async_copy(
                rows_v[b], table_sh.at[idx_v.at[t]], ssem[b])

        load_desc(0, 0).start()

        def pair(tt, carry):
            for b in range(2):
                t = tt * 2 + b
                load_desc(t, b).wait()
                pltpu.async_copy(
                    rows_v[b], table_sh.at[idx_v.at[t]], ssem[b], add=True)

                @pl.when(t >= 1)
                def _():
                    scat_desc(t - 1, 1 - b).wait()

                @pl.when(t + 1 < TRIPS)
                def _():
                    load_desc(t + 1, 1 - b).start()
            return carry

        lax.fori_loop(0, TRIPS // 2, pair, 0)
        scat_desc(TRIPS - 1, 1).wait()
        plsc.subcore_barrier()

        for k in range(RPT // CH):
            r0 = pl.multiple_of(sid * RPT + k * CH, CH)
            pltpu.async_copy(
                table_sh.at[pl.ds(r0, CH)], out_hbm.at[cid, pl.ds(r0, CH)], l0)
        for k in range(RPT // CH):
            r0 = pl.multiple_of(sid * RPT + k * CH, CH)
            pltpu.make_async_copy(
                table_sh.at[pl.ds(r0, CH)], out_hbm.at[cid, pl.ds(r0, CH)],
                l0).wait()

    return run(h, idx2d)


def _sc_gather(p, idx2d):
    mesh = plsc.VectorSubcoreMesh(core_axis_name="c", subcore_axis_name="s")

    @functools.partial(
        pl.kernel,
        mesh=mesh,
        out_type=jax.ShapeDtypeStruct((EP, H), jnp.float32),
        scratch_types=[
            pltpu.VMEM((TRIPS, CH), jnp.int32),
            pltpu.VMEM((CH, H), jnp.float32),
            pltpu.VMEM((CH, H), jnp.float32),
            pltpu.VMEM((CH, H), jnp.float32),
            pltpu.VMEM((CH, H), jnp.float32),
            pltpu.SemaphoreType.DMA,
            pltpu.SemaphoreType.DMA,
            pltpu.SemaphoreType.DMA,
            pltpu.SemaphoreType.DMA,
            pltpu.SemaphoreType.DMA,
            pltpu.SemaphoreType.DMA,
            pltpu.SemaphoreType.DMA,
            pltpu.SemaphoreType.DMA,
        ],
    )
    def run(p_hbm, idx_hbm, out_hbm, idx_v, r0v, r1v, r2v, r3v,
            g0, g1, g2, g3, w0, w1, w2, w3):
        cid = lax.axis_index("c")
        sid = lax.axis_index("s")
        wid = cid * 16 + sid
        start = pl.multiple_of(wid * TRIPS, 8)
        rows_v = (r0v, r1v, r2v, r3v)
        gsem = (g0, g1, g2, g3)
        wsem = (w0, w1, w2, w3)

        pltpu.sync_copy(idx_hbm.at[pl.ds(start, TRIPS)], idx_v)

        def gat_desc(t, b):
            return pltpu.make_async_copy(
                p_hbm.at[idx_v.at[t]], rows_v[b], gsem[b])

        def wb_desc(t, b):
            return pltpu.make_async_copy(
                rows_v[b],
                out_hbm.at[pl.ds(pl.multiple_of((start + t) * CH, CH), CH)],
                wsem[b])

        def quad(tt, carry):
            for b in range(4):
                t = tt * 4 + b

                @pl.when(t >= 4)
                def _():
                    wb_desc(t - 4, b).wait()

                gat_desc(t, b).start()

                @pl.when(t >= 1)
                def _():
                    gat_desc(t - 1, (b + 3) % 4).wait()
                    wb_desc(t - 1, (b + 3) % 4).start()
            return carry

        lax.fori_loop(0, TRIPS // 4, quad, 0)
        gat_desc(TRIPS - 1, 3).wait()
        wb_desc(TRIPS - 1, 3).start()
        for b in range(4):
            wb_desc(TRIPS - 4 + b, b).wait()

    return run(p, idx2d)


def _tc_p(parts, w2):
    def body(parts_ref, w2_ref, out_ref):
        psum = parts_ref[0] + parts_ref[1]
        out_ref[...] = lax.dot_general(
            psum, w2_ref[...], (((1,), (1,)), ((), ())),
            preferred_element_type=jnp.float32)

    return pl.pallas_call(
        body,
        out_shape=jax.ShapeDtypeStruct((NPAD, H), jnp.float32),
    )(parts, w2)


def _tc_step(h, h0, psrc, w2):
    nb = 50
    bs = E // nb

    def body(hrev_ref, h0_ref, psrc_ref, w2_ref, out_ref):
        q = lax.dot_general(
            hrev_ref[...], w2_ref[...], (((1,), (1,)), ((), ())),
            preferred_element_type=jnp.float32)
        out_ref[...] = jnp.maximum(h0_ref[...] + psrc_ref[...] - q, 0.0)

    return pl.pallas_call(
        body,
        grid=(nb,),
        in_specs=[
            pl.BlockSpec((bs, H), lambda i: ((i + nb // 2) % nb, 0)),
            pl.BlockSpec((bs, H), lambda i: (i, 0)),
            pl.BlockSpec((bs, H), lambda i: (i, 0)),
            pl.BlockSpec((H, H), lambda i: (0, 0)),
        ],
        out_specs=pl.BlockSpec((bs, H), lambda i: (i, 0)),
        out_shape=jax.ShapeDtypeStruct((E, H), jnp.float32),
    )(h, h0, psrc, w2)


def _tc_final(parts, x, batch2d, w3x, w3v, wh1, bh1, wh2, bh2):
    def body(parts_ref, x_ref, b_ref, w3x_ref, w3v_ref, wh1_ref, bh1_ref,
             wh2_ref, bh2_ref, out_ref):
        v = parts_ref[0, :N, :] + parts_ref[1, :N, :]
        na = jnp.maximum(
            lax.dot_general(x_ref[...], w3x_ref[...], (((1,), (1,)), ((), ())),
                            preferred_element_type=jnp.float32)
            + lax.dot_general(v, w3v_ref[...], (((1,), (1,)), ((), ())),
                              preferred_element_type=jnp.float32),
            0.0)
        gid = lax.broadcasted_iota(jnp.int32, (G, N), 0)
        onehot = (b_ref[...] == gid).astype(jnp.float32)
        g = lax.dot_general(onehot, na, (((1,), (0,)), ((), ())),
                            preferred_element_type=jnp.float32)
        t1 = jnp.maximum(
            lax.dot_general(g, wh1_ref[...], (((1,), (1,)), ((), ())),
                            preferred_element_type=jnp.float32)
            + bh1_ref[...], 0.0)
        out_ref[...] = lax.dot_general(
            t1, wh2_ref[...], (((1,), (1,)), ((), ())),
            preferred_element_type=jnp.float32) + bh2_ref[...]

    return pl.pallas_call(
        body,
        out_shape=jax.ShapeDtypeStruct((G, OUT), jnp.float32),
    )(parts, x, batch2d, w3x, w3v, wh1, bh1, wh2, bh2)


def kernel(x, edge_index, edge_attr, batch, W2, W3, Wh1, bh1, Wh2, bh2):
    src2d = edge_index[0].astype(jnp.int32).reshape(ROWS, CH)
    dst2d = edge_index[1].astype(jnp.int32).reshape(ROWS, CH)
    spread = (jnp.arange(PROWS * CH, dtype=jnp.int32) % N).reshape(PROWS, CH)
    src = jnp.where(
        (jnp.arange(PROWS, dtype=jnp.int32) < ROWS)[:, None],
        jnp.pad(src2d, ((0, PROWS - ROWS), (0, 0))), spread)
    dst = jnp.pad(dst2d, ((0, PROWS - ROWS), (0, 0)),
                  constant_values=JUNK_ROW)
    h0 = edge_attr

    h = h0
    for _ in range(2):
        parts = _sc_scatter(h, dst)
        p = _tc_p(parts, W2)
        psrc = _sc_gather(p, src)
        h = _tc_step(h, h0, psrc, W2)

    parts = _sc_scatter(h, dst)
    out = _tc_final(
        parts, x, batch.astype(jnp.int32).reshape(1, N),
        W3[:, :H], W3[:, H:], Wh1, bh1.reshape(1, H), Wh2,
        bh2.reshape(1, OUT))
    return out
